# Initial kernel scaffold; baseline (speedup 1.0000x reference)
import jax
import jax.numpy as jnp
from jax.experimental import pallas as pl


def kernel(codes, W0, W1):
    raise NotImplementedError("write your pallas kernel here")


# speedup vs baseline: 4.3214x; 4.3214x over previous
import functools

import jax
import jax.numpy as jnp
from jax import lax
from jax.experimental import pallas as pl
from jax.experimental.pallas import tpu as pltpu
from jax.experimental.pallas import tpu_sc as plsc

B = 1024
SEQ = 800
HID = 64
SPLIT = 200
NUM_WORKERS = 32
ROWS_PER_WORKER = B // NUM_WORKERS
CHUNK = 40


def kernel(codes, W0, W1):
    codes = codes.astype(jnp.int32)
    mesh = plsc.VectorSubcoreMesh(core_axis_name="c", subcore_axis_name="s")

    @functools.partial(
        pl.kernel,
        mesh=mesh,
        out_type=jax.ShapeDtypeStruct((B, SEQ, HID), jnp.float32),
        compiler_params=pltpu.CompilerParams(use_tc_tiling_on_sc=False),
        scratch_types=[
            pltpu.VMEM((SEQ,), jnp.int32),
            pltpu.VMEM((SEQ, HID), jnp.float32),
            pltpu.SemaphoreType.DMA,
        ],
    )
    def run(codes_hbm, w0_hbm, w1_hbm, out_hbm, idx_v, buf_v, sem):
        wid = lax.axis_index("s") * 2 + lax.axis_index("c")
        zeros16 = jnp.zeros((16,), jnp.float32)

        def body(i, carry):
            b = wid * ROWS_PER_WORKER + i
            pltpu.sync_copy(codes_hbm.at[b], idx_v)
            copies = []
            for c in range(SEQ // CHUNK):
                start = c * CHUNK
                tbl = w0_hbm if start < SPLIT else w1_hbm
                copies.append(
                    pltpu.async_copy(
                        tbl.at[idx_v.at[pl.ds(start, CHUNK)]],
                        buf_v.at[pl.ds(start, CHUNK)],
                        sem,
                    )
                )

            def scan_body(ci, mn):
                idx16 = idx_v[pl.ds(ci * 16, 16)]
                for j in range(16):
                    mn = jnp.minimum(mn, idx16[j])
                return mn

            mn = lax.fori_loop(0, SEQ // 16, scan_body, jnp.int32(1))

            for cp in copies:
                cp.wait()

            @pl.when(mn == 0)
            def _():
                def fix_body(ci, c2):
                    base = ci * 16
                    idx16 = idx_v[pl.ds(base, 16)]
                    for j in range(16):
                        @pl.when(idx16[j] == 0)
                        def _():
                            for k in range(HID // 16):
                                buf_v[base + j, pl.ds(k * 16, 16)] = zeros16

                    return c2

                lax.fori_loop(0, SEQ // 16, fix_body, 0)

            pltpu.sync_copy(buf_v, out_hbm.at[b])
            return carry

        lax.fori_loop(0, ROWS_PER_WORKER, body, 0)

    return run(codes, W0, W1)


# speedup vs baseline: 4.5518x; 1.0533x over previous
import functools

import jax
import jax.numpy as jnp
from jax import lax
from jax.experimental import pallas as pl
from jax.experimental.pallas import tpu as pltpu
from jax.experimental.pallas import tpu_sc as plsc

B = 1024
SEQ = 800
HID = 64
SPLIT = 200
NUM_WORKERS = 32
ROWS_PER_WORKER = B // NUM_WORKERS
HALF = ROWS_PER_WORKER // 2
CHUNKS = ((0, 128), (128, 72), (200, 128), (328, 128), (456, 128), (584, 128), (712, 88))


def kernel(codes, W0, W1):
    codes = codes.astype(jnp.int32)
    mesh = plsc.VectorSubcoreMesh(core_axis_name="c", subcore_axis_name="s")

    @functools.partial(
        pl.kernel,
        mesh=mesh,
        out_type=jax.ShapeDtypeStruct((B, SEQ, HID), jnp.float32),
        compiler_params=pltpu.CompilerParams(use_tc_tiling_on_sc=False),
        scratch_types=[
            pltpu.VMEM((SEQ,), jnp.int32),
            pltpu.VMEM((SEQ,), jnp.int32),
            pltpu.VMEM((SEQ, HID), jnp.float32),
            pltpu.VMEM((SEQ, HID), jnp.float32),
            pltpu.SemaphoreType.DMA,
            pltpu.SemaphoreType.DMA,
            pltpu.SemaphoreType.DMA,
            pltpu.SemaphoreType.DMA,
            pltpu.SemaphoreType.DMA,
            pltpu.SemaphoreType.DMA,
        ],
    )
    def run(codes_hbm, w0_hbm, w1_hbm, out_hbm,
            idx_a, idx_b, buf_a, buf_b,
            sem_ga, sem_gb, sem_oa, sem_ob, sem_ia, sem_ib):
        wid = lax.axis_index("s") * 2 + lax.axis_index("c")
        b0 = wid * ROWS_PER_WORKER
        zeros16 = jnp.zeros((16,), jnp.float32)

        def fire_gathers(idx_ref, buf_ref, sem):
            for start, ln in CHUNKS:
                tbl = w0_hbm if start < SPLIT else w1_hbm
                pltpu.async_copy(
                    tbl.at[idx_ref.at[pl.ds(start, ln)]],
                    buf_ref.at[pl.ds(start, ln)],
                    sem,
                )

        def drain(dummy_src, dst_ref, sem):
            pltpu.make_async_copy(dummy_src, dst_ref, sem).wait()

        def scan_row(idx_ref):
            def sb(ci, acc):
                return jnp.minimum(acc, idx_ref[pl.ds(ci * 16, 16)])

            accv = lax.fori_loop(0, SEQ // 16, sb, jnp.full((16,), 1, jnp.int32))
            mn = accv[0]
            for j in range(1, 16):
                mn = jnp.minimum(mn, accv[j])
            return mn

        def fix_row(idx_ref, buf_ref):
            def fb(ci, c2):
                idx16 = idx_ref[pl.ds(ci * 16, 16)]
                for j in range(16):
                    @pl.when(idx16[j] == 0)
                    def _():
                        for k in range(HID // 16):
                            buf_ref[ci * 16 + j, pl.ds(k * 16, 16)] = zeros16

                return c2

            lax.fori_loop(0, SEQ // 16, fb, 0)

        pltpu.sync_copy(codes_hbm.at[b0], idx_a)
        fire_gathers(idx_a, buf_a, sem_ga)
        pltpu.async_copy(codes_hbm.at[b0 + 1], idx_b, sem_ib)

        def body(g, carry):
            ra = b0 + 2 * g

            mna = scan_row(idx_a)
            drain(codes_hbm.at[b0], idx_b, sem_ib)

            @pl.when(g > 0)
            def _():
                drain(out_hbm.at[b0], buf_b, sem_ob)

            fire_gathers(idx_b, buf_b, sem_gb)
            drain(out_hbm.at[b0], buf_a, sem_ga)

            @pl.when(mna == 0)
            def _():
                fix_row(idx_a, buf_a)

            @pl.when(g < HALF - 1)
            def _():
                pltpu.async_copy(codes_hbm.at[ra + 2], idx_a, sem_ia)

            pltpu.async_copy(buf_a, out_hbm.at[ra], sem_oa)

            mnb = scan_row(idx_b)

            @pl.when(g < HALF - 1)
            def _():
                drain(codes_hbm.at[b0], idx_a, sem_ia)
                drain(out_hbm.at[b0], buf_a, sem_oa)
                fire_gathers(idx_a, buf_a, sem_ga)

            drain(out_hbm.at[b0], buf_b, sem_gb)

            @pl.when(mnb == 0)
            def _():
                fix_row(idx_b, buf_b)

            @pl.when(g < HALF - 1)
            def _():
                pltpu.async_copy(codes_hbm.at[ra + 3], idx_b, sem_ib)

            pltpu.async_copy(buf_b, out_hbm.at[ra + 1], sem_ob)
            return carry

        lax.fori_loop(0, HALF, body, 0)

        drain(out_hbm.at[b0], buf_a, sem_oa)
        drain(out_hbm.at[b0], buf_b, sem_ob)

    return run(codes, W0, W1)
